# Initial kernel scaffold; baseline (speedup 1.0000x reference)
import jax
import jax.numpy as jnp
from jax.experimental import pallas as pl


def kernel(inputs, table):
    raise NotImplementedError("write your pallas kernel here")


# speedup vs baseline: 2.5829x; 2.5829x over previous
import functools

import jax
import jax.numpy as jnp
from jax import lax
from jax.experimental import pallas as pl
from jax.experimental.pallas import tpu as pltpu
from jax.experimental.pallas import tpu_sc as plsc

_DIM = 64
_SCALE = 10.0

_info = plsc.get_sparse_core_info()
_NC, _NS = _info.num_cores, _info.num_subcores
_NW = _NC * _NS

_G = 128


def _scale_body(t_ref, o_ref):
    o_ref[...] = t_ref[...] * _SCALE


def _prescale(table):
    rows = table.shape[0]
    br = 2000
    return pl.pallas_call(
        _scale_body,
        grid=(rows // br,),
        in_specs=[pl.BlockSpec((br, _DIM), lambda i: (i, 0))],
        out_specs=pl.BlockSpec((br, _DIM), lambda i: (i, 0)),
        out_shape=jax.ShapeDtypeStruct((rows, _DIM), jnp.float32),
    )(table)


@functools.lru_cache(maxsize=None)
def _make_gather(b_total):
    assert b_total % (_NW * _G) == 0
    n_it = b_total // (_NW * _G)
    b_per_w = n_it * _G
    mesh = plsc.VectorSubcoreMesh(core_axis_name="c", subcore_axis_name="s")

    @functools.partial(
        pl.kernel,
        mesh=mesh,
        out_type=jax.ShapeDtypeStruct((b_total, _DIM), jnp.float32),
        compiler_params=pltpu.CompilerParams(use_tc_tiling_on_sc=False),
        scratch_types=[
            pltpu.VMEM((n_it, _G), jnp.int32),
            pltpu.VMEM((_G, _DIM), jnp.float32),
            pltpu.SemaphoreType.DMA,
        ],
    )
    def k(table_hbm, idx_hbm, out_hbm, idx_v, rows_v, sem):
        wid = lax.axis_index("s") * _NC + lax.axis_index("c")
        base = wid * b_per_w
        pltpu.sync_copy(idx_hbm.at[wid], idx_v)

        def body(j, carry):
            pltpu.async_copy(table_hbm.at[idx_v.at[j]], rows_v, sem).wait()
            pltpu.sync_copy(rows_v, out_hbm.at[pl.ds(base + j * _G, _G)])
            return carry

        lax.fori_loop(0, n_it, body, 0)

    return k


def kernel(inputs, table):
    b_total = inputs.size
    idx = inputs.reshape(_NW, b_total // (_NW * _G), _G)
    scaled = _prescale(table)
    out = _make_gather(b_total)(scaled, idx)
    return out.reshape(inputs.shape + (_DIM,))


# speedup vs baseline: 2.6746x; 1.0355x over previous
import functools

import jax
import jax.numpy as jnp
from jax import lax
from jax.experimental import pallas as pl
from jax.experimental.pallas import tpu as pltpu
from jax.experimental.pallas import tpu_sc as plsc

_DIM = 64
_SCALE = 10.0

_info = plsc.get_sparse_core_info()
_NC, _NS = _info.num_cores, _info.num_subcores
_NW = _NC * _NS

_G = 128


def _scale_body(t_ref, o_ref):
    o_ref[...] = t_ref[...] * _SCALE


def _prescale(table):
    rows = table.shape[0]
    br = 2000
    return pl.pallas_call(
        _scale_body,
        grid=(rows // br,),
        in_specs=[pl.BlockSpec((br, _DIM), lambda i: (i, 0))],
        out_specs=pl.BlockSpec((br, _DIM), lambda i: (i, 0)),
        out_shape=jax.ShapeDtypeStruct((rows, _DIM), jnp.float32),
    )(table)


@functools.lru_cache(maxsize=None)
def _make_gather(b_total):
    assert b_total % (_NW * _G) == 0
    n_it = b_total // (_NW * _G)
    b_per_w = n_it * _G
    mesh = plsc.VectorSubcoreMesh(core_axis_name="c", subcore_axis_name="s")

    assert n_it % 2 == 0

    @functools.partial(
        pl.kernel,
        mesh=mesh,
        out_type=jax.ShapeDtypeStruct((b_total, _DIM), jnp.float32),
        compiler_params=pltpu.CompilerParams(use_tc_tiling_on_sc=False),
        scratch_types=[
            pltpu.VMEM((n_it, _G), jnp.int32),
            pltpu.VMEM((2, _G, _DIM), jnp.float32),
            pltpu.SemaphoreType.DMA,
            pltpu.SemaphoreType.DMA,
            pltpu.SemaphoreType.DMA,
            pltpu.SemaphoreType.DMA,
        ],
    )
    def k(table_hbm, idx_hbm, out_hbm, idx_v, rows, gsem0, gsem1, osem0, osem1):
        gsems = (gsem0, gsem1)
        osems = (osem0, osem1)
        wid = lax.axis_index("s") * _NC + lax.axis_index("c")
        base = wid * b_per_w
        pltpu.sync_copy(idx_hbm.at[wid], idx_v)
        pltpu.async_copy(table_hbm.at[idx_v.at[0]], rows.at[0], gsems[0])

        def grp(g, carry):
            for b in range(2):
                j = g * 2 + b
                nb = 1 - b
                pltpu.make_async_copy(
                    table_hbm.at[idx_v.at[j]], rows.at[b], gsems[b]
                ).wait()
                pltpu.async_copy(
                    rows.at[b], out_hbm.at[pl.ds(base + j * _G, _G)], osems[b]
                )

                @pl.when(j >= 1)
                def _():
                    pltpu.make_async_copy(
                        rows.at[nb], out_hbm.at[pl.ds(base, _G)], osems[nb]
                    ).wait()

                @pl.when(j + 1 < n_it)
                def _():
                    pltpu.async_copy(
                        table_hbm.at[idx_v.at[j + 1]], rows.at[nb], gsems[nb]
                    )

            return carry

        lax.fori_loop(0, n_it // 2, grp, 0)
        lb = (n_it - 1) % 2
        pltpu.make_async_copy(
            rows.at[lb], out_hbm.at[pl.ds(base, _G)], osems[lb]
        ).wait()

    return k


def kernel(inputs, table):
    b_total = inputs.size
    idx = inputs.reshape(_NW, b_total // (_NW * _G), _G)
    scaled = _prescale(table)
    out = _make_gather(b_total)(scaled, idx)
    return out.reshape(inputs.shape + (_DIM,))


# speedup vs baseline: 3.3574x; 1.2553x over previous
import functools

import jax
import jax.numpy as jnp
from jax import lax
from jax.experimental import pallas as pl
from jax.experimental.pallas import tpu as pltpu
from jax.experimental.pallas import tpu_sc as plsc

_DIM = 64
_SCALE = 10.0

_info = plsc.get_sparse_core_info()
_NC, _NS = _info.num_cores, _info.num_subcores
_NW = _NC * _NS

_G = 128


@functools.lru_cache(maxsize=None)
def _make_gather(b_total):
    assert b_total % (_NW * _G) == 0
    n_it = b_total // (_NW * _G)
    assert n_it % 2 == 0
    b_per_w = n_it * _G
    mesh = plsc.VectorSubcoreMesh(core_axis_name="c", subcore_axis_name="s")

    @functools.partial(
        pl.kernel,
        mesh=mesh,
        out_type=jax.ShapeDtypeStruct((b_total, _DIM), jnp.float32),
        compiler_params=pltpu.CompilerParams(use_tc_tiling_on_sc=False),
        scratch_types=[
            pltpu.VMEM((b_per_w,), jnp.int32),
            pltpu.VMEM((2, _G, _DIM), jnp.float32),
            pltpu.SemaphoreType.DMA,
            pltpu.SemaphoreType.DMA,
            pltpu.SemaphoreType.DMA,
            pltpu.SemaphoreType.DMA,
        ],
    )
    def k(table_hbm, idx_hbm, out_hbm, idx_v, rows, gsem0, gsem1, osem0, osem1):
        gsems = (gsem0, gsem1)
        osems = (osem0, osem1)
        wid = lax.axis_index("s") * _NC + lax.axis_index("c")
        base = wid * b_per_w
        pltpu.sync_copy(idx_hbm.at[pl.ds(base, b_per_w)], idx_v)
        pltpu.async_copy(
            table_hbm.at[idx_v.at[pl.ds(0, _G)]], rows.at[0], gsems[0]
        )

        def grp(g, carry):
            for b in range(2):
                j = g * 2 + b
                nb = 1 - b
                pltpu.make_async_copy(
                    table_hbm.at[idx_v.at[pl.ds(0, _G)]], rows.at[b], gsems[b]
                ).wait()

                @pl.when(j >= 1)
                def _():
                    pltpu.make_async_copy(
                        rows.at[nb], out_hbm.at[pl.ds(base, _G)], osems[nb]
                    ).wait()

                @pl.when(j + 1 < n_it)
                def _():
                    pltpu.async_copy(
                        table_hbm.at[idx_v.at[pl.ds((j + 1) * _G, _G)]],
                        rows.at[nb],
                        gsems[nb],
                    )

                @plsc.parallel_loop(0, _G, step=1, unroll=4)
                def _(r):
                    for c in range(_DIM // 16):
                        sl = (b, r, pl.ds(c * 16, 16))
                        rows[sl] = rows[sl] * _SCALE

                pltpu.async_copy(
                    rows.at[b], out_hbm.at[pl.ds(base + j * _G, _G)], osems[b]
                )

            return carry

        lax.fori_loop(0, n_it // 2, grp, 0)
        lb = (n_it - 1) % 2
        pltpu.make_async_copy(
            rows.at[lb], out_hbm.at[pl.ds(base, _G)], osems[lb]
        ).wait()

    return k


def kernel(inputs, table):
    b_total = inputs.size
    idx = inputs.reshape(b_total)
    out = _make_gather(b_total)(table, idx)
    return out.reshape(inputs.shape + (_DIM,))


# speedup vs baseline: 3.3630x; 1.0017x over previous
import functools

import jax
import jax.numpy as jnp
from jax import lax
from jax.experimental import pallas as pl
from jax.experimental.pallas import tpu as pltpu
from jax.experimental.pallas import tpu_sc as plsc

_DIM = 64
_SCALE = 10.0

_info = plsc.get_sparse_core_info()
_NC, _NS = _info.num_cores, _info.num_subcores
_NW = _NC * _NS

_G = 128


@functools.lru_cache(maxsize=None)
def _make_gather(b_total):
    assert b_total % (_NW * _G) == 0
    n_it = b_total // (_NW * _G)
    assert n_it % 2 == 0
    b_per_w = n_it * _G
    mesh = plsc.VectorSubcoreMesh(core_axis_name="c", subcore_axis_name="s")

    chunk = _G * _DIM

    @functools.partial(
        pl.kernel,
        mesh=mesh,
        out_type=jax.ShapeDtypeStruct((b_total * _DIM,), jnp.float32),
        compiler_params=pltpu.CompilerParams(use_tc_tiling_on_sc=False),
        scratch_types=[
            pltpu.VMEM((b_per_w,), jnp.int32),
            pltpu.VMEM((2, _G, _DIM), jnp.float32),
            pltpu.VMEM((2, _G * _DIM), jnp.float32),
            pltpu.SemaphoreType.DMA,
            pltpu.SemaphoreType.DMA,
            pltpu.SemaphoreType.DMA,
            pltpu.SemaphoreType.DMA,
        ],
    )
    def k(table_hbm, idx_hbm, out_hbm, idx_v, rows, flat, gsem0, gsem1,
          osem0, osem1):
        gsems = (gsem0, gsem1)
        osems = (osem0, osem1)
        wid = lax.axis_index("s") * _NC + lax.axis_index("c")
        base = wid * b_per_w
        pltpu.sync_copy(idx_hbm.at[pl.ds(base, b_per_w)], idx_v)
        pltpu.async_copy(
            table_hbm.at[idx_v.at[pl.ds(0, _G)]], rows.at[0], gsems[0]
        )

        def grp(g, carry):
            for b in range(2):
                j = g * 2 + b
                nb = 1 - b
                pltpu.make_async_copy(
                    table_hbm.at[idx_v.at[pl.ds(0, _G)]], rows.at[b], gsems[b]
                ).wait()

                @pl.when(j + 1 < n_it)
                def _():
                    pltpu.async_copy(
                        table_hbm.at[idx_v.at[pl.ds((j + 1) * _G, _G)]],
                        rows.at[nb],
                        gsems[nb],
                    )

                @pl.when(j >= 2)
                def _():
                    pltpu.make_async_copy(
                        flat.at[b], out_hbm.at[pl.ds(0, chunk)], osems[b]
                    ).wait()

                @plsc.parallel_loop(0, _G * _DIM // 16, step=1, unroll=8)
                def _(p):
                    r = p // (_DIM // 16)
                    c = (p % (_DIM // 16)) * 16
                    flat[b, pl.ds(p * 16, 16)] = (
                        rows[b, r, pl.ds(c, 16)] * _SCALE
                    )

                pltpu.async_copy(
                    flat.at[b],
                    out_hbm.at[pl.ds((base + j * _G) * _DIM, chunk)],
                    osems[b],
                )

            return carry

        lax.fori_loop(0, n_it // 2, grp, 0)
        for b in range(2):
            pltpu.make_async_copy(
                flat.at[b], out_hbm.at[pl.ds(0, chunk)], osems[b]
            ).wait()

    return k


def kernel(inputs, table):
    b_total = inputs.size
    idx = inputs.reshape(b_total)
    out = _make_gather(b_total)(table, idx)
    return out.reshape(inputs.shape + (_DIM,))


# speedup vs baseline: 3.5191x; 1.0464x over previous
import functools

import jax
import jax.numpy as jnp
from jax import lax
from jax.experimental import pallas as pl
from jax.experimental.pallas import tpu as pltpu
from jax.experimental.pallas import tpu_sc as plsc

_DIM = 64
_SCALE = 10.0

_info = plsc.get_sparse_core_info()
_NC, _NS = _info.num_cores, _info.num_subcores
_NW = _NC * _NS

_OC = 4
_G1 = 104


@functools.lru_cache(maxsize=None)
def _make_gather(n_rows, seq):
    flat_per_chunk = _OC * seq
    assert n_rows % _NW == 0
    rows_per_w = n_rows // _NW
    assert rows_per_w % _OC == 0
    n_it = rows_per_w // _OC
    assert n_it % 2 == 0
    b_per_w = rows_per_w * seq
    g2 = flat_per_chunk - _G1
    assert 0 < g2 <= 128 and _G1 <= 128 and _G1 % 8 == 0
    mesh = plsc.VectorSubcoreMesh(core_axis_name="c", subcore_axis_name="s")

    @functools.partial(
        pl.kernel,
        mesh=mesh,
        out_type=jax.ShapeDtypeStruct((n_rows, seq, _DIM), jnp.float32),
        compiler_params=pltpu.CompilerParams(use_tc_tiling_on_sc=False),
        scratch_types=[
            pltpu.VMEM((b_per_w,), jnp.int32),
            pltpu.VMEM((2, flat_per_chunk, _DIM), jnp.float32),
            pltpu.VMEM((2, _OC, seq, _DIM), jnp.float32),
            pltpu.SemaphoreType.DMA,
            pltpu.SemaphoreType.DMA,
            pltpu.SemaphoreType.DMA,
            pltpu.SemaphoreType.DMA,
        ],
    )
    def k(table_hbm, idx_hbm, out_hbm, idx_v, rows, boxes, gsem0, gsem1,
          osem0, osem1):
        gsems = (gsem0, gsem1)
        osems = (osem0, osem1)
        wid = lax.axis_index("s") * _NC + lax.axis_index("c")
        base = wid * b_per_w
        row0 = wid * rows_per_w
        pltpu.sync_copy(idx_hbm.at[pl.ds(base, b_per_w)], idx_v)

        def fire_gathers(j, b):
            off = j * flat_per_chunk
            pltpu.async_copy(
                table_hbm.at[idx_v.at[pl.ds(off, _G1)]],
                rows.at[b, pl.ds(0, _G1)],
                gsems[b],
            )
            pltpu.async_copy(
                table_hbm.at[idx_v.at[pl.ds(off + _G1, g2)]],
                rows.at[b, pl.ds(_G1, g2)],
                gsems[b],
            )

        def wait_gathers(b):
            pltpu.make_async_copy(
                table_hbm.at[idx_v.at[pl.ds(0, _G1)]],
                rows.at[b, pl.ds(0, _G1)],
                gsems[b],
            ).wait()
            pltpu.make_async_copy(
                table_hbm.at[idx_v.at[pl.ds(0, g2)]],
                rows.at[b, pl.ds(_G1, g2)],
                gsems[b],
            ).wait()

        fire_gathers(0, 0)

        def grp(g, carry):
            for b in range(2):
                j = g * 2 + b
                nb = 1 - b
                wait_gathers(b)

                @pl.when(j + 1 < n_it)
                def _():
                    fire_gathers(j + 1, nb)

                @pl.when(j >= 2)
                def _():
                    pltpu.make_async_copy(
                        boxes.at[b], out_hbm.at[pl.ds(0, _OC)], osems[b]
                    ).wait()

                for o in range(_OC):
                    @plsc.parallel_loop(0, seq * _DIM // 16, step=1, unroll=8)
                    def _(p):
                        r = p // (_DIM // 16)
                        c = (p % (_DIM // 16)) * 16
                        boxes[b, o, r, pl.ds(c, 16)] = (
                            rows[b, o * seq + r, pl.ds(c, 16)] * _SCALE
                        )

                pltpu.async_copy(
                    boxes.at[b],
                    out_hbm.at[pl.ds(row0 + j * _OC, _OC)],
                    osems[b],
                )

            return carry

        lax.fori_loop(0, n_it // 2, grp, 0)
        for b in range(2):
            pltpu.make_async_copy(
                boxes.at[b], out_hbm.at[pl.ds(0, _OC)], osems[b]
            ).wait()

    return k


def kernel(inputs, table):
    n_rows, seq = inputs.shape
    idx = inputs.reshape(inputs.size)
    return _make_gather(n_rows, seq)(table, idx)
